# Initial kernel scaffold; baseline (speedup 1.0000x reference)
import jax
import jax.numpy as jnp
from jax.experimental import pallas as pl


def kernel(x, e, edge_index, W, b):
    raise NotImplementedError("write your pallas kernel here")


# speedup vs baseline: 1.8354x; 1.8354x over previous
import functools

import jax
import jax.numpy as jnp
from jax import lax
from jax.experimental import pallas as pl
from jax.experimental.pallas import tpu as pltpu
from jax.experimental.pallas import tpu_sc as plsc

N = 100000
E = 6400000
DV = 128
DE = 4
DP = 8
DOUT = 128

NC = 2
NS = 16
NW = NC * NS
CHUNK = 2048
NCHUNKS = E // CHUNK
CHUNKS_PER_TILE = -(-NCHUNKS // NW)
NPAD = 100096
ROWS_PER_TILE = NPAD // NS


def _sc_scatter_body(dst_hbm, e_hbm, zero_hbm, out_hbm, idx_v, rows_v, acc_sh):
    c = lax.axis_index("c")
    s = lax.axis_index("s")
    w = c * NS + s

    pltpu.sync_copy(zero_hbm,
                    acc_sh.at[pl.ds(s * ROWS_PER_TILE, ROWS_PER_TILE), :])
    plsc.subcore_barrier()

    def chunk_step(k, carry):
        g = w + k * NW

        @pl.when(g < NCHUNKS)
        def _():
            pltpu.sync_copy(dst_hbm.at[pl.ds(g * CHUNK, CHUNK)], idx_v)
            pltpu.sync_copy(e_hbm.at[pl.ds(g * CHUNK, CHUNK), :], rows_v)
            pltpu.sync_copy(rows_v, acc_sh.at[idx_v], add=True)

        return carry

    lax.fori_loop(0, CHUNKS_PER_TILE, chunk_step, 0)
    plsc.subcore_barrier()

    pltpu.sync_copy(acc_sh.at[pl.ds(s * ROWS_PER_TILE, ROWS_PER_TILE), :],
                    out_hbm.at[c, pl.ds(s * ROWS_PER_TILE, ROWS_PER_TILE), :])


_sc_scatter = functools.partial(
    pl.kernel,
    out_type=jax.ShapeDtypeStruct((NC, NPAD, DP), jnp.float32),
    mesh=plsc.VectorSubcoreMesh(core_axis_name="c", subcore_axis_name="s"),
    scratch_types=[
        pltpu.VMEM((CHUNK,), jnp.int32),
        pltpu.VMEM((CHUNK, DP), jnp.float32),
        pltpu.VMEM_SHARED((NPAD, DP), jnp.float32),
    ],
    compiler_params=pltpu.CompilerParams(use_tc_tiling_on_sc=False),
)(_sc_scatter_body)


BR = 2000


def _tc_body(x_ref, p_ref, w1_ref, w2_ref, b_ref, o_ref):
    agg = p_ref[0] + p_ref[1]
    o_ref[...] = (
        jnp.dot(x_ref[...], w1_ref[...], preferred_element_type=jnp.float32)
        + jnp.dot(agg, w2_ref[...], preferred_element_type=jnp.float32)
        + b_ref[...]
    )


def _tc_matmul(x, partials, w1, w2p, b2):
    grid = (N // BR,)
    return pl.pallas_call(
        _tc_body,
        grid=grid,
        in_specs=[
            pl.BlockSpec((BR, DV), lambda i: (i, 0)),
            pl.BlockSpec((NC, BR, DP), lambda i: (0, i, 0)),
            pl.BlockSpec((DV, DOUT), lambda i: (0, 0)),
            pl.BlockSpec((DP, DOUT), lambda i: (0, 0)),
            pl.BlockSpec((1, DOUT), lambda i: (0, 0)),
        ],
        out_specs=pl.BlockSpec((BR, DOUT), lambda i: (i, 0)),
        out_shape=jax.ShapeDtypeStruct((N, DOUT), jnp.float32),
        compiler_params=pltpu.CompilerParams(
            dimension_semantics=("parallel",),
        ),
    )(x, partials, w1, w2p, b2)


def kernel(x, e, edge_index, W, b):
    dst = edge_index[1]
    e8 = jnp.concatenate([e, jnp.zeros((E, DP - DE), jnp.float32)], axis=1)
    zeros = jnp.zeros((ROWS_PER_TILE, DP), jnp.float32)
    partials = _sc_scatter(dst, e8, zeros)
    w1 = W[:DV]
    w2p = jnp.concatenate([W[DV:], jnp.zeros((DP - DE, DOUT), jnp.float32)],
                          axis=0)
    b2 = b.reshape(1, DOUT)
    return _tc_matmul(x, partials, w1, w2p, b2)


# speedup vs baseline: 2.0792x; 1.1328x over previous
import functools

import jax
import jax.numpy as jnp
from jax import lax
from jax.experimental import pallas as pl
from jax.experimental.pallas import tpu as pltpu
from jax.experimental.pallas import tpu_sc as plsc

N = 100000
E = 6400000
DV = 128
DE = 4
DP = 8
DOUT = 128

NC = 2
NS = 16
NW = NC * NS
CHUNK = 2048
NCHUNKS = E // CHUNK
CHUNKS_PER_TILE = -(-NCHUNKS // NW)
NPAD = 100096
ROWS_PER_TILE = NPAD // NS


def _sc_scatter_body(dst_hbm, e_hbm, zero_hbm, out_hbm, idx_v, rows_v, acc_sh):
    c = lax.axis_index("c")
    s = lax.axis_index("s")
    w = c * NS + s

    pltpu.sync_copy(zero_hbm,
                    acc_sh.at[pl.ds(s * ROWS_PER_TILE, ROWS_PER_TILE), :])
    plsc.subcore_barrier()

    def chunk_step(k, carry):
        g = w + k * NW

        @pl.when(g < NCHUNKS)
        def _():
            pltpu.sync_copy(dst_hbm.at[pl.ds(g * CHUNK, CHUNK)], idx_v)
            pltpu.sync_copy(e_hbm.at[pl.ds(g * CHUNK, CHUNK), :], rows_v)
            pltpu.sync_copy(rows_v, acc_sh.at[idx_v], add=True)

        return carry

    lax.fori_loop(0, CHUNKS_PER_TILE, chunk_step, 0)
    plsc.subcore_barrier()

    pltpu.sync_copy(acc_sh.at[pl.ds(s * ROWS_PER_TILE, ROWS_PER_TILE), :],
                    out_hbm.at[c, pl.ds(s * ROWS_PER_TILE, ROWS_PER_TILE), :])


_sc_scatter = functools.partial(
    pl.kernel,
    out_type=jax.ShapeDtypeStruct((NC, NPAD, DP), jnp.float32),
    mesh=plsc.VectorSubcoreMesh(core_axis_name="c", subcore_axis_name="s"),
    scratch_types=[
        pltpu.VMEM((CHUNK,), jnp.int32),
        pltpu.VMEM((CHUNK, DP), jnp.float32),
        pltpu.VMEM_SHARED((NPAD, DP), jnp.float32),
    ],
    compiler_params=pltpu.CompilerParams(use_tc_tiling_on_sc=False),
)(_sc_scatter_body)


EV = E // 32
BRV = 2000


def _pad_body(a_ref, m_ref, o_ref):
    o_ref[...] = jnp.dot(a_ref[...], m_ref[...],
                         preferred_element_type=jnp.float32)


def _tc_pad(e):
    a = e.reshape(EV, 128)
    lanes = jnp.arange(128)
    j = lanes % 8
    i = lanes // 8
    col = jnp.where(j < 4, i * 4 + j, 0)
    hot = (j < 4).astype(jnp.float32)
    m_half = (jnp.arange(64)[:, None] == col[None, :]) * hot[None, :]
    m2 = jnp.zeros((128, 256), jnp.float32)
    m2 = m2.at[:64, :128].set(m_half).at[64:, 128:].set(m_half)
    out = pl.pallas_call(
        _pad_body,
        grid=(EV // BRV,),
        in_specs=[
            pl.BlockSpec((BRV, 128), lambda i: (i, 0)),
            pl.BlockSpec((128, 256), lambda i: (0, 0)),
        ],
        out_specs=pl.BlockSpec((BRV, 256), lambda i: (i, 0)),
        out_shape=jax.ShapeDtypeStruct((EV, 256), jnp.float32),
        compiler_params=pltpu.CompilerParams(
            dimension_semantics=("parallel",),
        ),
    )(a, m2)
    return out.reshape(E, DP)


BR = 2000


def _tc_body(x_ref, p_ref, w1_ref, w2_ref, b_ref, o_ref):
    agg = p_ref[0] + p_ref[1]
    o_ref[...] = (
        jnp.dot(x_ref[...], w1_ref[...], preferred_element_type=jnp.float32)
        + jnp.dot(agg, w2_ref[...], preferred_element_type=jnp.float32)
        + b_ref[...]
    )


def _tc_matmul(x, partials, w1, w2p, b2):
    grid = (N // BR,)
    return pl.pallas_call(
        _tc_body,
        grid=grid,
        in_specs=[
            pl.BlockSpec((BR, DV), lambda i: (i, 0)),
            pl.BlockSpec((NC, BR, DP), lambda i: (0, i, 0)),
            pl.BlockSpec((DV, DOUT), lambda i: (0, 0)),
            pl.BlockSpec((DP, DOUT), lambda i: (0, 0)),
            pl.BlockSpec((1, DOUT), lambda i: (0, 0)),
        ],
        out_specs=pl.BlockSpec((BR, DOUT), lambda i: (i, 0)),
        out_shape=jax.ShapeDtypeStruct((N, DOUT), jnp.float32),
        compiler_params=pltpu.CompilerParams(
            dimension_semantics=("parallel",),
        ),
    )(x, partials, w1, w2p, b2)


def kernel(x, e, edge_index, W, b):
    dst = edge_index[1]
    e8 = _tc_pad(e)
    zeros = jnp.zeros((ROWS_PER_TILE, DP), jnp.float32)
    partials = _sc_scatter(dst, e8, zeros)
    w1 = W[:DV]
    w2p = jnp.concatenate([W[DV:], jnp.zeros((DP - DE, DOUT), jnp.float32)],
                          axis=0)
    b2 = b.reshape(1, DOUT)
    return _tc_matmul(x, partials, w1, w2p, b2)


# speedup vs baseline: 33.0418x; 15.8917x over previous
import functools

import jax
import jax.numpy as jnp
from jax import lax
from jax.experimental import pallas as pl
from jax.experimental.pallas import tpu as pltpu
from jax.experimental.pallas import tpu_sc as plsc

N = 100000
E = 6400000
DV = 128
DE = 4
DP = 8
DOUT = 128

NC = 2
NS = 16
NW = NC * NS
CHUNK = 2048
TPC = CHUNK // 128
NCHUNKS = E // CHUNK
CHUNKS_PER_TILE = -(-NCHUNKS // NW)
NPAD = 100096
ROWS_PER_TILE = NPAD // NS


def _sc_scatter_body(dst_hbm, e3_hbm, zacc_hbm, zrows_hbm, out_hbm,
                     idx_v, st_v, rows_v, acc_sh):
    c = lax.axis_index("c")
    s = lax.axis_index("s")
    w = c * NS + s

    pltpu.sync_copy(zacc_hbm,
                    acc_sh.at[pl.ds(s * ROWS_PER_TILE, ROWS_PER_TILE), :])
    pltpu.sync_copy(zrows_hbm, rows_v)
    plsc.subcore_barrier()

    lanes16 = lax.iota(jnp.int32, 16)

    def chunk_step(k, carry):
        g = w + k * NW

        @pl.when(g < NCHUNKS)
        def _():
            pltpu.sync_copy(dst_hbm.at[pl.ds(g * CHUNK, CHUNK)], idx_v)
            pltpu.sync_copy(e3_hbm.at[pl.ds(g * TPC, TPC), :, :], st_v)

            def tile_step(t, tcarry):
                for m in range(8):
                    ridx = t * 128 + m * 16 + lanes16
                    for f in range(DE):
                        x = st_v[t, f, pl.ds(m * 16, 16)]
                        plsc.store_scatter(
                            rows_v, [ridx, jnp.full((16,), f, jnp.int32)], x)
                return tcarry

            lax.fori_loop(0, TPC, tile_step, 0)
            pltpu.sync_copy(rows_v, acc_sh.at[idx_v], add=True)

        return carry

    lax.fori_loop(0, CHUNKS_PER_TILE, chunk_step, 0)
    plsc.subcore_barrier()

    pltpu.sync_copy(acc_sh.at[pl.ds(s * ROWS_PER_TILE, ROWS_PER_TILE), :],
                    out_hbm.at[c, pl.ds(s * ROWS_PER_TILE, ROWS_PER_TILE), :])


_sc_scatter = functools.partial(
    pl.kernel,
    out_type=jax.ShapeDtypeStruct((NC, NPAD, DP), jnp.float32),
    mesh=plsc.VectorSubcoreMesh(core_axis_name="c", subcore_axis_name="s"),
    scratch_types=[
        pltpu.VMEM((CHUNK,), jnp.int32),
        pltpu.VMEM((TPC, DE, 128), jnp.float32),
        pltpu.VMEM((CHUNK, DP), jnp.float32),
        pltpu.VMEM_SHARED((NPAD, DP), jnp.float32),
    ],
    compiler_params=pltpu.CompilerParams(use_tc_tiling_on_sc=False,
                                         needs_layout_passes=False),
)(_sc_scatter_body)


BR = 2000


def _tc_body(x_ref, p_ref, w1_ref, w2_ref, b_ref, o_ref):
    agg = p_ref[0] + p_ref[1]
    o_ref[...] = (
        jnp.dot(x_ref[...], w1_ref[...], preferred_element_type=jnp.float32)
        + jnp.dot(agg, w2_ref[...], preferred_element_type=jnp.float32)
        + b_ref[...]
    )


def _tc_matmul(x, partials, w1, w2p, b2):
    grid = (N // BR,)
    return pl.pallas_call(
        _tc_body,
        grid=grid,
        in_specs=[
            pl.BlockSpec((BR, DV), lambda i: (i, 0)),
            pl.BlockSpec((NC, BR, DP), lambda i: (0, i, 0)),
            pl.BlockSpec((DV, DOUT), lambda i: (0, 0)),
            pl.BlockSpec((DP, DOUT), lambda i: (0, 0)),
            pl.BlockSpec((1, DOUT), lambda i: (0, 0)),
        ],
        out_specs=pl.BlockSpec((BR, DOUT), lambda i: (i, 0)),
        out_shape=jax.ShapeDtypeStruct((N, DOUT), jnp.float32),
        compiler_params=pltpu.CompilerParams(
            dimension_semantics=("parallel",),
        ),
    )(x, partials, w1, w2p, b2)


def kernel(x, e, edge_index, W, b):
    dst = edge_index[1]
    e3 = e.reshape(E // 128, 128, DE).transpose(0, 2, 1)
    zacc = jnp.zeros((ROWS_PER_TILE, DP), jnp.float32)
    zrows = jnp.zeros((CHUNK, DP), jnp.float32)
    partials = _sc_scatter(dst, e3, zacc, zrows)
    w1 = W[:DV]
    w2p = jnp.concatenate([W[DV:], jnp.zeros((DP - DE, DOUT), jnp.float32)],
                          axis=0)
    b2 = b.reshape(1, DOUT)
    return _tc_matmul(x, partials, w1, w2p, b2)


# speedup vs baseline: 44.5613x; 1.3486x over previous
import functools

import jax
import jax.numpy as jnp
from jax import lax
from jax.experimental import pallas as pl
from jax.experimental.pallas import tpu as pltpu
from jax.experimental.pallas import tpu_sc as plsc

N = 100000
E = 6400000
DV = 128
DE = 4
DP = 8
DOUT = 128

NC = 2
NS = 16
NW = NC * NS
CHUNK = 2048
TPC = CHUNK // 128
NCHUNKS = E // CHUNK
CHUNKS_PER_TILE = -(-NCHUNKS // NW)
NPAD = 100096
ROWS_PER_TILE = NPAD // NS


def _sc_scatter_body(dst_hbm, e3_hbm, zacc_hbm, zrows_hbm, out_hbm,
                     idx_v0, st_v0, idx_v1, st_v1, rows_v, acc_sh,
                     sem0, sem1):
    c = lax.axis_index("c")
    s = lax.axis_index("s")
    w = c * NS + s

    pltpu.sync_copy(zacc_hbm,
                    acc_sh.at[pl.ds(s * ROWS_PER_TILE, ROWS_PER_TILE), :])
    pltpu.sync_copy(zrows_hbm, rows_v)
    plsc.subcore_barrier()

    lanes16 = lax.iota(jnp.int32, 16)

    def stage_start(k, idx_v, st_v, sem):
        g = w + k * NW

        @pl.when(g < NCHUNKS)
        def _():
            pltpu.async_copy(dst_hbm.at[pl.ds(g * CHUNK, CHUNK)], idx_v, sem)
            pltpu.async_copy(e3_hbm.at[pl.ds(g * TPC, TPC), :, :], st_v, sem)

    def stage_wait(k, idx_v, st_v, sem):
        g = w + k * NW

        @pl.when(g < NCHUNKS)
        def _():
            pltpu.make_async_copy(dst_hbm.at[pl.ds(g * CHUNK, CHUNK)],
                                  idx_v, sem).wait()
            pltpu.make_async_copy(e3_hbm.at[pl.ds(g * TPC, TPC), :, :],
                                  st_v, sem).wait()

    def process(k, idx_v, st_v):
        g = w + k * NW

        @pl.when(g < NCHUNKS)
        def _():
            def tile_step(t, tcarry):
                for m in range(8):
                    ridx = t * 128 + m * 16 + lanes16
                    for f in range(DE):
                        x = st_v[t, f, pl.ds(m * 16, 16)]
                        plsc.store_scatter(
                            rows_v, [ridx, jnp.full((16,), f, jnp.int32)], x)
                return tcarry

            lax.fori_loop(0, TPC, tile_step, 0)
            pltpu.sync_copy(rows_v, acc_sh.at[idx_v], add=True)

    stage_start(0, idx_v0, st_v0, sem0)

    def chunk_pair(t, carry):
        k0 = 2 * t
        stage_start(k0 + 1, idx_v1, st_v1, sem1)
        stage_wait(k0, idx_v0, st_v0, sem0)
        process(k0, idx_v0, st_v0)
        stage_start(k0 + 2, idx_v0, st_v0, sem0)
        stage_wait(k0 + 1, idx_v1, st_v1, sem1)
        process(k0 + 1, idx_v1, st_v1)
        return carry

    lax.fori_loop(0, CHUNKS_PER_TILE // 2, chunk_pair, 0)
    plsc.subcore_barrier()

    pltpu.sync_copy(acc_sh.at[pl.ds(s * ROWS_PER_TILE, ROWS_PER_TILE), :],
                    out_hbm.at[c, pl.ds(s * ROWS_PER_TILE, ROWS_PER_TILE), :])


_sc_scatter = functools.partial(
    pl.kernel,
    out_type=jax.ShapeDtypeStruct((NC, NPAD, DP), jnp.float32),
    mesh=plsc.VectorSubcoreMesh(core_axis_name="c", subcore_axis_name="s"),
    scratch_types=[
        pltpu.VMEM((CHUNK,), jnp.int32),
        pltpu.VMEM((TPC, DE, 128), jnp.float32),
        pltpu.VMEM((CHUNK,), jnp.int32),
        pltpu.VMEM((TPC, DE, 128), jnp.float32),
        pltpu.VMEM((CHUNK, DP), jnp.float32),
        pltpu.VMEM_SHARED((NPAD, DP), jnp.float32),
        pltpu.SemaphoreType.DMA,
        pltpu.SemaphoreType.DMA,
    ],
    compiler_params=pltpu.CompilerParams(use_tc_tiling_on_sc=False,
                                         needs_layout_passes=False),
)(_sc_scatter_body)


BR = 2000


def _tc_body(x_ref, p_ref, w1_ref, w2_ref, b_ref, o_ref):
    agg = p_ref[0] + p_ref[1]
    o_ref[...] = (
        jnp.dot(x_ref[...], w1_ref[...], preferred_element_type=jnp.float32)
        + jnp.dot(agg, w2_ref[...], preferred_element_type=jnp.float32)
        + b_ref[...]
    )


def _tc_matmul(x, partials, w1, w2p, b2):
    grid = (N // BR,)
    return pl.pallas_call(
        _tc_body,
        grid=grid,
        in_specs=[
            pl.BlockSpec((BR, DV), lambda i: (i, 0)),
            pl.BlockSpec((NC, BR, DP), lambda i: (0, i, 0)),
            pl.BlockSpec((DV, DOUT), lambda i: (0, 0)),
            pl.BlockSpec((DP, DOUT), lambda i: (0, 0)),
            pl.BlockSpec((1, DOUT), lambda i: (0, 0)),
        ],
        out_specs=pl.BlockSpec((BR, DOUT), lambda i: (i, 0)),
        out_shape=jax.ShapeDtypeStruct((N, DOUT), jnp.float32),
        compiler_params=pltpu.CompilerParams(
            dimension_semantics=("parallel",),
        ),
    )(x, partials, w1, w2p, b2)


def kernel(x, e, edge_index, W, b):
    dst = edge_index[1]
    e3 = e.reshape(E // 128, 128, DE).transpose(0, 2, 1)
    zacc = jnp.zeros((ROWS_PER_TILE, DP), jnp.float32)
    zrows = jnp.zeros((CHUNK, DP), jnp.float32)
    partials = _sc_scatter(dst, e3, zacc, zrows)
    w1 = W[:DV]
    w2p = jnp.concatenate([W[DV:], jnp.zeros((DP - DE, DOUT), jnp.float32)],
                          axis=0)
    b2 = b.reshape(1, DOUT)
    return _tc_matmul(x, partials, w1, w2p, b2)
